# baseline (speedup 1.0000x reference)
import functools

import jax
import jax.numpy as jnp
from jax import lax
from jax.experimental import pallas as pl
from jax.experimental.pallas import tpu as pltpu
from jax.experimental.pallas import tpu_sc as plsc

BATCH = 16384
EMB_DIM = 64
FC_OUT = 32
X_COLS = 472

NC, NS = 2, 16
NW = NC * NS
B_PER_W = BATCH // NW
CHUNK = 128
N_CHUNK = B_PER_W // CHUNK


def _sc_gather(emb, x):
    mesh = plsc.VectorSubcoreMesh(
        core_axis_name="c", subcore_axis_name="s",
        num_cores=NC, num_subcores=NS)

    @functools.partial(
        pl.kernel,
        out_type=jax.ShapeDtypeStruct((BATCH, EMB_DIM), jnp.float32),
        mesh=mesh,
        compiler_params=pltpu.CompilerParams(
            use_tc_tiling_on_sc=False, needs_layout_passes=False),
        scratch_types=[
            pltpu.VMEM((B_PER_W, 8), jnp.float32),
            pltpu.VMEM((B_PER_W,), jnp.int32),
            pltpu.VMEM((B_PER_W, EMB_DIM), jnp.float32),
            pltpu.SemaphoreType.DMA,
        ],
    )
    def gather_kernel(table_hbm, x_hbm, out_hbm,
                      colval_v, idx_v, rows_v, sem):
        wid = lax.axis_index("s") * NC + lax.axis_index("c")
        base = wid * B_PER_W
        lane = lax.iota(jnp.int32, 16)
        zeros16 = jnp.zeros((16,), jnp.int32)
        pltpu.sync_copy(x_hbm.at[pl.ds(base, B_PER_W), pl.ds(0, 8)],
                        colval_v)
        for k in range(B_PER_W // 16):
            vals = plsc.load_gather(colval_v, [lane + k * 16, zeros16])
            idx_v[pl.ds(k * 16, 16)] = vals.astype(jnp.int32)
        row_copies = [
            pltpu.async_copy(
                table_hbm.at[idx_v.at[pl.ds(j * CHUNK, CHUNK)]],
                rows_v.at[pl.ds(j * CHUNK, CHUNK)],
                sem)
            for j in range(N_CHUNK)
        ]
        for c in row_copies:
            c.wait()
        pltpu.sync_copy(rows_v, out_hbm.at[pl.ds(base, B_PER_W)])

    return gather_kernel(emb, x)


def _tc_fused(x, embedded, wx, we, b1):
    bm = 1024

    def body(x_ref, e_ref, wx_ref, we_ref, b_ref, o_ref):
        acc = jnp.dot(x_ref[...], wx_ref[...],
                      preferred_element_type=jnp.float32)
        acc = acc + jnp.dot(e_ref[...], we_ref[...],
                            preferred_element_type=jnp.float32)
        o_ref[...] = jnp.maximum(acc + b_ref[...], 0.0)

    return pl.pallas_call(
        body,
        grid=(BATCH // bm,),
        in_specs=[
            pl.BlockSpec((bm, X_COLS), lambda i: (i, 0)),
            pl.BlockSpec((bm, EMB_DIM), lambda i: (i, 0)),
            pl.BlockSpec((X_COLS, FC_OUT), lambda i: (0, 0)),
            pl.BlockSpec((EMB_DIM, FC_OUT), lambda i: (0, 0)),
            pl.BlockSpec((1, FC_OUT), lambda i: (0, 0)),
        ],
        out_specs=pl.BlockSpec((bm, FC_OUT), lambda i: (i, 0)),
        out_shape=jax.ShapeDtypeStruct((BATCH, FC_OUT), jnp.float32),
    )(x, embedded, wx, we, b1)


def kernel(x, emb, W1, b1):
    embedded = _sc_gather(emb, x)
    wx = jnp.concatenate([jnp.zeros((1, FC_OUT), W1.dtype), W1[EMB_DIM:]],
                         axis=0)
    we = W1[:EMB_DIM]
    return _tc_fused(x, embedded, wx, we, b1.reshape(1, FC_OUT))


# speedup vs baseline: 1.0191x; 1.0191x over previous
import functools

import jax
import jax.numpy as jnp
from jax import lax
from jax.experimental import pallas as pl
from jax.experimental.pallas import tpu as pltpu
from jax.experimental.pallas import tpu_sc as plsc

BATCH = 16384
EMB_DIM = 64
FC_OUT = 32
X_COLS = 472

NC, NS = 2, 16
NW = NC * NS
B_PER_W = BATCH // NW
CHUNK = 128
N_CHUNK = B_PER_W // CHUNK
IDX_ROWS = BATCH // CHUNK


def _tc_stage1(x, wx, b1):
    bm = 2048

    def body(x_ref, wx_ref, b_ref, p_ref, idx_ref):
        xb = x_ref[...]
        p_ref[...] = jnp.dot(xb, wx_ref[...],
                             preferred_element_type=jnp.float32) + b_ref[...]
        col = xb[:, 0:1].astype(jnp.int32)
        idx_ref[...] = col.reshape(bm // CHUNK, CHUNK)

    return pl.pallas_call(
        body,
        grid=(BATCH // bm,),
        in_specs=[
            pl.BlockSpec((bm, X_COLS), lambda i: (i, 0)),
            pl.BlockSpec((X_COLS, FC_OUT), lambda i: (0, 0)),
            pl.BlockSpec((1, FC_OUT), lambda i: (0, 0)),
        ],
        out_specs=[
            pl.BlockSpec((bm, FC_OUT), lambda i: (i, 0)),
            pl.BlockSpec((bm // CHUNK, CHUNK), lambda i: (i, 0)),
        ],
        out_shape=[
            jax.ShapeDtypeStruct((BATCH, FC_OUT), jnp.float32),
            jax.ShapeDtypeStruct((IDX_ROWS, CHUNK), jnp.int32),
        ],
    )(x, wx, b1)


def _sc_gather(emb, idx2):
    mesh = plsc.VectorSubcoreMesh(
        core_axis_name="c", subcore_axis_name="s",
        num_cores=NC, num_subcores=NS)

    @functools.partial(
        pl.kernel,
        out_type=jax.ShapeDtypeStruct((BATCH, EMB_DIM), jnp.float32),
        mesh=mesh,
        compiler_params=pltpu.CompilerParams(use_tc_tiling_on_sc=False),
        scratch_types=[
            pltpu.VMEM((N_CHUNK, CHUNK), jnp.int32),
            pltpu.VMEM((B_PER_W, EMB_DIM), jnp.float32),
            pltpu.SemaphoreType.DMA,
        ],
    )
    def gather_kernel(table_hbm, idx_hbm, out_hbm, idx_v, rows_v, sem):
        wid = lax.axis_index("s") * NC + lax.axis_index("c")
        pltpu.sync_copy(idx_hbm.at[pl.ds(wid * N_CHUNK, N_CHUNK)], idx_v)
        row_copies = [
            pltpu.async_copy(
                table_hbm.at[idx_v.at[j]],
                rows_v.at[pl.ds(j * CHUNK, CHUNK)],
                sem)
            for j in range(N_CHUNK)
        ]
        for c in row_copies:
            c.wait()
        pltpu.sync_copy(rows_v, out_hbm.at[pl.ds(wid * B_PER_W, B_PER_W)])

    return gather_kernel(emb, idx2)


def _tc_stage2(p, embedded, we):
    bm = 2048

    def body(p_ref, e_ref, we_ref, o_ref):
        acc = p_ref[...] + jnp.dot(e_ref[...], we_ref[...],
                                   preferred_element_type=jnp.float32)
        o_ref[...] = jnp.maximum(acc, 0.0)

    return pl.pallas_call(
        body,
        grid=(BATCH // bm,),
        in_specs=[
            pl.BlockSpec((bm, FC_OUT), lambda i: (i, 0)),
            pl.BlockSpec((bm, EMB_DIM), lambda i: (i, 0)),
            pl.BlockSpec((EMB_DIM, FC_OUT), lambda i: (0, 0)),
        ],
        out_specs=pl.BlockSpec((bm, FC_OUT), lambda i: (i, 0)),
        out_shape=jax.ShapeDtypeStruct((BATCH, FC_OUT), jnp.float32),
    )(p, embedded, we)


def kernel(x, emb, W1, b1):
    wx = jnp.concatenate([jnp.zeros((1, FC_OUT), W1.dtype), W1[EMB_DIM:]],
                         axis=0)
    p, idx2 = _tc_stage1(x, wx, b1.reshape(1, FC_OUT))
    embedded = _sc_gather(emb, idx2)
    return _tc_stage2(p, embedded, W1[:EMB_DIM])
